# baseline (speedup 1.0000x reference)
import functools
import jax
import jax.numpy as jnp
from jax import lax
from jax.experimental import pallas as pl
from jax.experimental.pallas import tpu as pltpu
from jax.experimental.pallas import tpu_sc as plsc

NC = 2
NS = 16
LN = 16
NW = NC * NS
CKW = 128
CKS = 1024
STG = 128
RB = 1024
NGROUPS = 16

_mesh = plsc.VectorSubcoreMesh(
    core_axis_name="c", subcore_axis_name="s", num_cores=NC, num_subcores=NS)
_sc_params = pltpu.CompilerParams(use_tc_tiling_on_sc=False)


def _make_sc_deg(NP, EP, ck):
    cpt = EP // (NW * ck)
    rows = NP // NS

    @functools.partial(
        pl.kernel, mesh=_mesh, compiler_params=_sc_params,
        out_type=jax.ShapeDtypeStruct((NC, NP), jnp.float32),
        scratch_types=[
            pltpu.VMEM((cpt, ck), jnp.int32),
            pltpu.VMEM((cpt, ck), jnp.float32),
            pltpu.VMEM((rows,), jnp.float32),
            pltpu.VMEM_SHARED((NP,), jnp.float32),
        ],
    )
    def k(dst_hbm, w_hbm, z_hbm, out_hbm, didx, wv, stage, acc):
        c = lax.axis_index("c")
        s = lax.axis_index("s")
        cbase = (c * NS + s) * cpt
        pltpu.sync_copy(z_hbm, stage)
        pltpu.sync_copy(stage, acc.at[pl.ds(s * rows, rows)])
        pltpu.sync_copy(dst_hbm.at[pl.ds(cbase, cpt)], didx)
        pltpu.sync_copy(w_hbm.at[pl.ds(cbase, cpt)], wv)
        plsc.subcore_barrier()

        def chunk(j, carry):
            pltpu.sync_copy(wv.at[j], acc.at[didx.at[j]], add=True)
            return carry

        lax.fori_loop(0, cpt, chunk, 0)
        plsc.subcore_barrier()
        pltpu.sync_copy(acc.at[pl.ds(s * rows, rows)], stage)
        pltpu.sync_copy(stage, out_hbm.at[c, pl.ds(s * rows, rows)])

    return k


def _make_sc_mp(NP, EP, F, ck):
    cpt = EP // (NW * ck)
    assert cpt % 2 == 0
    rows = NP // NS

    @functools.partial(
        pl.kernel, mesh=_mesh, compiler_params=_sc_params,
        out_type=jax.ShapeDtypeStruct((NC, NP, F), jnp.float32),
        scratch_types=[
            pltpu.VMEM((cpt, ck), jnp.int32),
            pltpu.VMEM((cpt, ck), jnp.int32),
            pltpu.VMEM((cpt, ck), jnp.float32),
            pltpu.VMEM((ck, F), jnp.float32),
            pltpu.VMEM((ck, F), jnp.float32),
            pltpu.VMEM((ck, F), jnp.float32),
            pltpu.VMEM((ck, F), jnp.float32),
            pltpu.VMEM((STG, F), jnp.float32),
            pltpu.VMEM_SHARED((NP, F), jnp.float32),
            pltpu.SemaphoreType.DMA,
            pltpu.SemaphoreType.DMA,
            pltpu.SemaphoreType.DMA,
        ],
    )
    def k(g_hbm, src_hbm, dst_hbm, w_hbm, z_hbm, out_hbm,
          sidx, didx, wv, rb0, rb1, sb0, sb1, stage, acc,
          gs0, gs1, ss):
        c = lax.axis_index("c")
        s = lax.axis_index("s")
        cbase = (c * NS + s) * cpt
        pltpu.sync_copy(z_hbm, stage)
        for r in range(rows // STG):
            pltpu.sync_copy(stage, acc.at[pl.ds(s * rows + r * STG, STG)])
        pltpu.sync_copy(src_hbm.at[pl.ds(cbase, cpt)], sidx)
        pltpu.sync_copy(dst_hbm.at[pl.ds(cbase, cpt)], didx)
        pltpu.sync_copy(w_hbm.at[pl.ds(cbase, cpt)], wv)
        plsc.subcore_barrier()

        def gather(j, rb, gs):
            pltpu.async_copy(g_hbm.at[sidx.at[j]], rb, gs)

        def gather_wait(j, rb, gs):
            pltpu.make_async_copy(g_hbm.at[sidx.at[j]], rb, gs).wait()

        def scale(j, rb, sb):
            def egrp(i, ecarry):
                wv16 = wv[j, pl.ds(i * LN, LN)]
                for j16 in range(LN):
                    bw = jnp.full((LN,), wv16[j16], jnp.float32)
                    e = i * LN + j16
                    for f in range(F // LN):
                        sl = pl.ds(f * LN, LN)
                        sb[e, sl] = rb[e, sl] * bw
                return ecarry

            lax.fori_loop(0, ck // LN, egrp, 0, unroll=True)

        def scatter(j, sb, ss):
            pltpu.async_copy(sb, acc.at[didx.at[j]], ss, add=True)

        def scatter_wait(j, sb, ss):
            pltpu.make_async_copy(sb, acc.at[didx.at[j]], ss).wait()

        gather(0, rb0, gs0)
        gather(1, rb1, gs1)
        gather_wait(0, rb0, gs0)
        scale(0, rb0, sb0)
        scatter(0, sb0, ss)
        gather(2, rb0, gs0)
        gather_wait(1, rb1, gs1)
        scale(1, rb1, sb1)
        scatter(1, sb1, ss)
        gather(3, rb1, gs1)

        def pair(jj, carry):
            j0 = 2 * jj
            gather_wait(j0, rb0, gs0)
            scatter_wait(j0 - 2, sb0, ss)
            scale(j0, rb0, sb0)
            scatter(j0, sb0, ss)

            @pl.when(j0 + 2 < cpt)
            def _():
                gather(j0 + 2, rb0, gs0)

            gather_wait(j0 + 1, rb1, gs1)
            scatter_wait(j0 - 1, sb1, ss)
            scale(j0 + 1, rb1, sb1)
            scatter(j0 + 1, sb1, ss)

            @pl.when(j0 + 3 < cpt)
            def _():
                gather(j0 + 3, rb1, gs1)

            return carry

        lax.fori_loop(1, cpt // 2, pair, 0)
        scatter_wait(cpt - 2, sb0, ss)
        scatter_wait(cpt - 1, sb1, ss)
        plsc.subcore_barrier()
        for r in range(rows // STG):
            pltpu.sync_copy(acc.at[pl.ds(s * rows + r * STG, STG)], stage)
            pltpu.sync_copy(stage, out_hbm.at[c, pl.ds(s * rows + r * STG, STG)])

    return k


def _make_sc_mp1(NP, EP, ck):
    cpt = EP // (NW * ck)
    assert cpt % 2 == 0
    rows = NP // NS

    @functools.partial(
        pl.kernel, mesh=_mesh, compiler_params=_sc_params,
        out_type=jax.ShapeDtypeStruct((NC, NP), jnp.float32),
        scratch_types=[
            pltpu.VMEM((cpt, ck), jnp.int32),
            pltpu.VMEM((cpt, ck), jnp.int32),
            pltpu.VMEM((cpt, ck), jnp.float32),
            pltpu.VMEM((ck,), jnp.float32),
            pltpu.VMEM((ck,), jnp.float32),
            pltpu.VMEM((rows,), jnp.float32),
            pltpu.VMEM_SHARED((NP,), jnp.float32),
            pltpu.SemaphoreType.DMA,
            pltpu.SemaphoreType.DMA,
        ],
    )
    def k(g_hbm, src_hbm, dst_hbm, w_hbm, z_hbm, out_hbm,
          sidx, didx, wv, vb0, vb1, stage, acc, sem0, sem1):
        c = lax.axis_index("c")
        s = lax.axis_index("s")
        cbase = (c * NS + s) * cpt
        pltpu.sync_copy(z_hbm, stage)
        pltpu.sync_copy(stage, acc.at[pl.ds(s * rows, rows)])
        pltpu.sync_copy(src_hbm.at[pl.ds(cbase, cpt)], sidx)
        pltpu.sync_copy(dst_hbm.at[pl.ds(cbase, cpt)], didx)
        pltpu.sync_copy(w_hbm.at[pl.ds(cbase, cpt)], wv)
        plsc.subcore_barrier()

        def scale_scatter(j, vb):
            for i in range(ck // LN):
                sl = pl.ds(i * LN, LN)
                vb[sl] = vb[sl] * wv[j, sl]
            pltpu.sync_copy(vb, acc.at[didx.at[j]], add=True)

        pltpu.async_copy(g_hbm.at[sidx.at[0]], vb0, sem0)

        def pair(jj, carry):
            j0 = 2 * jj
            pltpu.make_async_copy(g_hbm.at[sidx.at[j0]], vb0, sem0).wait()
            pltpu.async_copy(g_hbm.at[sidx.at[j0 + 1]], vb1, sem1)
            scale_scatter(j0, vb0)
            pltpu.make_async_copy(g_hbm.at[sidx.at[j0 + 1]], vb1, sem1).wait()

            @pl.when(jj + 1 < cpt // 2)
            def _():
                pltpu.async_copy(g_hbm.at[sidx.at[j0 + 2]], vb0, sem0)

            scale_scatter(j0 + 1, vb1)
            return carry

        lax.fori_loop(0, cpt // 2, pair, 0)
        plsc.subcore_barrier()
        pltpu.sync_copy(acc.at[pl.ds(s * rows, rows)], stage)
        pltpu.sync_copy(stage, out_hbm.at[c, pl.ds(s * rows, rows)])

    return k


def _tc_first(NP, D, F):
    def body(deg_ref, x_ref, w_ref, dis_ref, g_ref):
        deg = deg_ref[0] + deg_ref[1] + 1.0
        dis = lax.rsqrt(deg)
        dis_ref[...] = dis
        g_ref[...] = jnp.dot(x_ref[...], w_ref[...],
                             preferred_element_type=jnp.float32) * dis

    return pl.pallas_call(
        body,
        grid=(NP // RB,),
        in_specs=[
            pl.BlockSpec((NC, RB, 1), lambda i: (0, i, 0)),
            pl.BlockSpec((RB, D), lambda i: (i, 0)),
            pl.BlockSpec((D, F), lambda i: (0, 0)),
        ],
        out_specs=[
            pl.BlockSpec((RB, 1), lambda i: (i, 0)),
            pl.BlockSpec((RB, F), lambda i: (i, 0)),
        ],
        out_shape=[
            jax.ShapeDtypeStruct((NP, 1), jnp.float32),
            jax.ShapeDtypeStruct((NP, F), jnp.float32),
        ],
    )


def _tc_mid(NP, Fin, Fout):
    split = Fout == 128

    def body(s_ref, g_ref, dis_ref, b_ref, w_ref, *out_refs):
        dis = dis_ref[...]
        h = dis * (s_ref[0] + s_ref[1] + g_ref[...]) + b_ref[...]
        h = jnp.maximum(h, 0.0)
        g = jnp.dot(h, w_ref[...], preferred_element_type=jnp.float32) * dis
        if split:
            out_refs[0][...] = g[:, :64]
            out_refs[1][...] = g[:, 64:]
        else:
            out_refs[0][...] = g

    if split:
        out_specs = [pl.BlockSpec((RB, 64), lambda i: (i, 0)),
                     pl.BlockSpec((RB, 64), lambda i: (i, 0))]
        out_shape = [jax.ShapeDtypeStruct((NP, 64), jnp.float32),
                     jax.ShapeDtypeStruct((NP, 64), jnp.float32)]
    else:
        out_specs = pl.BlockSpec((RB, Fout), lambda i: (i, 0))
        out_shape = jax.ShapeDtypeStruct((NP, Fout), jnp.float32)

    return pl.pallas_call(
        body,
        grid=(NP // RB,),
        in_specs=[
            pl.BlockSpec((NC, RB, Fin), lambda i: (0, i, 0)),
            pl.BlockSpec((RB, Fin), lambda i: (i, 0)),
            pl.BlockSpec((RB, 1), lambda i: (i, 0)),
            pl.BlockSpec((1, Fin), lambda i: (0, 0)),
            pl.BlockSpec((Fin, Fout), lambda i: (0, 0)),
        ],
        out_specs=out_specs,
        out_shape=out_shape,
    )


def _tc_mid2(NP, Fout):
    def body(sa_ref, sb_ref, ga_ref, gb_ref, dis_ref, b_ref, w_ref, out_ref):
        dis = dis_ref[...]
        ha = dis * (sa_ref[0] + sa_ref[1] + ga_ref[...])
        hb = dis * (sb_ref[0] + sb_ref[1] + gb_ref[...])
        h = jnp.concatenate([ha, hb], axis=1) + b_ref[...]
        h = jnp.maximum(h, 0.0)
        out_ref[...] = jnp.dot(h, w_ref[...],
                               preferred_element_type=jnp.float32) * dis

    return pl.pallas_call(
        body,
        grid=(NP // RB,),
        in_specs=[
            pl.BlockSpec((NC, RB, 64), lambda i: (0, i, 0)),
            pl.BlockSpec((NC, RB, 64), lambda i: (0, i, 0)),
            pl.BlockSpec((RB, 64), lambda i: (i, 0)),
            pl.BlockSpec((RB, 64), lambda i: (i, 0)),
            pl.BlockSpec((RB, 1), lambda i: (i, 0)),
            pl.BlockSpec((1, 128), lambda i: (0, 0)),
            pl.BlockSpec((128, Fout), lambda i: (0, 0)),
        ],
        out_specs=pl.BlockSpec((RB, Fout), lambda i: (i, 0)),
        out_shape=jax.ShapeDtypeStruct((NP, Fout), jnp.float32),
    )


def _tc_last(NP, G):
    def body(s_ref, g_ref, dis_ref, b_ref, batch_ref, out_ref):
        h = dis_ref[...] * (s_ref[0] + s_ref[1] + g_ref[...]) + b_ref[...]
        gids = lax.broadcasted_iota(jnp.int32, (1, G), 1)
        onehot = (batch_ref[...] == gids).astype(jnp.float32)
        dn = (((0,), (0,)), ((), ()))
        sums = lax.dot_general(onehot, h, dn,
                               preferred_element_type=jnp.float32)
        ones = jnp.ones((NP, 1), jnp.float32)
        cnts = lax.dot_general(onehot, ones, dn,
                               preferred_element_type=jnp.float32)
        out_ref[...] = sums / jnp.maximum(cnts, 1.0)

    return pl.pallas_call(
        body,
        in_specs=[
            pl.BlockSpec((NC, NP, 1), lambda: (0, 0, 0)),
            pl.BlockSpec((NP, 1), lambda: (0, 0)),
            pl.BlockSpec((NP, 1), lambda: (0, 0)),
            pl.BlockSpec((1, 1), lambda: (0, 0)),
            pl.BlockSpec((NP, 1), lambda: (0, 0)),
        ],
        out_specs=pl.BlockSpec((G, 1), lambda: (0, 0)),
        out_shape=jax.ShapeDtypeStruct((G, 1), jnp.float32),
    )


def kernel(x, edge_index, edge_weight, batch,
           W1, b1, W2, b2, W3, b3, W4, b4):
    N, D = x.shape
    E = edge_index.shape[1]
    G = NGROUPS
    f32 = jnp.float32

    NP = ((N + 1 + RB - 1) // RB) * RB
    EP = -(-E // (2 * NW * CKS)) * (2 * NW * CKS)
    rows = NP // NS

    pad_dst = N + (jnp.arange(EP - E, dtype=jnp.int32) % (NP - N))
    pad_src = jnp.arange(EP - E, dtype=jnp.int32) % N
    src_f = jnp.concatenate([edge_index[0], pad_src])
    dst_f = jnp.concatenate([edge_index[1], pad_dst])
    w_f = jnp.concatenate([edge_weight, jnp.zeros((EP - E,), f32)])
    src_w = src_f.reshape(EP // CKW, CKW)
    dst_w = dst_f.reshape(EP // CKW, CKW)
    w_w = w_f.reshape(EP // CKW, CKW)
    src_s = src_f.reshape(EP // CKS, CKS)
    dst_s = dst_f.reshape(EP // CKS, CKS)
    w_s = w_f.reshape(EP // CKS, CKS)
    x_p = jnp.concatenate([x, jnp.zeros((NP - N, D), f32)])
    batch_p = jnp.concatenate([batch, jnp.full((NP - N,), G, jnp.int32)])[:, None]
    z1 = jnp.zeros((rows,), f32)
    z64 = jnp.zeros((STG, 64), f32)

    sc_deg = _make_sc_deg(NP, EP, CKS)
    sc_mp64 = _make_sc_mp(NP, EP, 64, CKW)
    sc_mp1 = _make_sc_mp1(NP, EP, CKS)

    deg_part = sc_deg(dst_s, w_s, z1)
    dis, g1 = _tc_first(NP, D, 64)(deg_part[..., None], x_p, W1)
    s1 = sc_mp64(g1, src_w, dst_w, w_w, z64)
    g2a, g2b = _tc_mid(NP, 64, 128)(s1, g1, dis, b1[None, :], W2)
    s2a = sc_mp64(g2a, src_w, dst_w, w_w, z64)
    s2b = sc_mp64(g2b, src_w, dst_w, w_w, z64)
    g3 = _tc_mid2(NP, 64)(s2a, s2b, g2a, g2b, dis, b2[None, :], W3)
    s3 = sc_mp64(g3, src_w, dst_w, w_w, z64)
    g4 = _tc_mid(NP, 64, 1)(s3, g3, dis, b3[None, :], W4)
    s4 = sc_mp1(g4[:, 0], src_s, dst_s, w_s, z1)
    out = _tc_last(NP, G)(s4[..., None], g4, dis, b4[None, :], batch_p)
    return out


# speedup vs baseline: 1.0080x; 1.0080x over previous
import functools
import jax
import jax.numpy as jnp
from jax import lax
from jax.experimental import pallas as pl
from jax.experimental.pallas import tpu as pltpu
from jax.experimental.pallas import tpu_sc as plsc

NC = 2
NS = 16
LN = 16
NW = NC * NS
CKW = 128
CKS = 512
STG = 128
RB = 1024
NGROUPS = 16

_mesh = plsc.VectorSubcoreMesh(
    core_axis_name="c", subcore_axis_name="s", num_cores=NC, num_subcores=NS)
_sc_params = pltpu.CompilerParams(use_tc_tiling_on_sc=False)


def _make_sc_deg(NP, EP, ck):
    cpt = EP // (NW * ck)
    rows = NP // NS

    @functools.partial(
        pl.kernel, mesh=_mesh, compiler_params=_sc_params,
        out_type=jax.ShapeDtypeStruct((NC, NP), jnp.float32),
        scratch_types=[
            pltpu.VMEM((cpt, ck), jnp.int32),
            pltpu.VMEM((cpt, ck), jnp.float32),
            pltpu.VMEM((rows,), jnp.float32),
            pltpu.VMEM_SHARED((NP,), jnp.float32),
        ],
    )
    def k(dst_hbm, w_hbm, z_hbm, out_hbm, didx, wv, stage, acc):
        c = lax.axis_index("c")
        s = lax.axis_index("s")
        cbase = (c * NS + s) * cpt
        pltpu.sync_copy(z_hbm, stage)
        pltpu.sync_copy(stage, acc.at[pl.ds(s * rows, rows)])
        pltpu.sync_copy(dst_hbm.at[pl.ds(cbase, cpt)], didx)
        pltpu.sync_copy(w_hbm.at[pl.ds(cbase, cpt)], wv)
        plsc.subcore_barrier()

        def chunk(j, carry):
            pltpu.sync_copy(wv.at[j], acc.at[didx.at[j]], add=True)
            return carry

        lax.fori_loop(0, cpt, chunk, 0)
        plsc.subcore_barrier()
        pltpu.sync_copy(acc.at[pl.ds(s * rows, rows)], stage)
        pltpu.sync_copy(stage, out_hbm.at[c, pl.ds(s * rows, rows)])

    return k


def _make_sc_mp(NP, EP, F, ck):
    cpt = EP // (NW * ck)
    assert cpt % 2 == 0
    rows = NP // NS

    @functools.partial(
        pl.kernel, mesh=_mesh, compiler_params=_sc_params,
        out_type=jax.ShapeDtypeStruct((NC, NP, F), jnp.float32),
        scratch_types=[
            pltpu.VMEM((cpt, ck), jnp.int32),
            pltpu.VMEM((cpt, ck), jnp.int32),
            pltpu.VMEM((cpt, ck), jnp.float32),
            pltpu.VMEM((ck, F), jnp.float32),
            pltpu.VMEM((ck, F), jnp.float32),
            pltpu.VMEM((ck, F), jnp.float32),
            pltpu.VMEM((ck, F), jnp.float32),
            pltpu.VMEM((STG, F), jnp.float32),
            pltpu.VMEM_SHARED((NP, F), jnp.float32),
            pltpu.SemaphoreType.DMA,
            pltpu.SemaphoreType.DMA,
            pltpu.SemaphoreType.DMA,
        ],
    )
    def k(g_hbm, src_hbm, dst_hbm, w_hbm, z_hbm, out_hbm,
          sidx, didx, wv, rb0, rb1, sb0, sb1, stage, acc,
          gs0, gs1, ss):
        c = lax.axis_index("c")
        s = lax.axis_index("s")
        cbase = (c * NS + s) * cpt
        pltpu.sync_copy(z_hbm, acc.at[pl.ds(s * rows, rows)])
        pltpu.sync_copy(src_hbm.at[pl.ds(cbase, cpt)], sidx)
        pltpu.sync_copy(dst_hbm.at[pl.ds(cbase, cpt)], didx)
        pltpu.sync_copy(w_hbm.at[pl.ds(cbase, cpt)], wv)
        plsc.subcore_barrier()

        def gather(j, rb, gs):
            pltpu.async_copy(g_hbm.at[sidx.at[j]], rb, gs)

        def gather_wait(j, rb, gs):
            pltpu.make_async_copy(g_hbm.at[sidx.at[j]], rb, gs).wait()

        def scale(j, rb, sb):
            def egrp(i, ecarry):
                wv16 = wv[j, pl.ds(i * LN, LN)]
                for j16 in range(LN):
                    bw = jnp.full((LN,), wv16[j16], jnp.float32)
                    e = i * LN + j16
                    for f in range(F // LN):
                        sl = pl.ds(f * LN, LN)
                        sb[e, sl] = rb[e, sl] * bw
                return ecarry

            lax.fori_loop(0, ck // LN, egrp, 0)

        def scatter(j, sb, ss):
            pltpu.async_copy(sb, acc.at[didx.at[j]], ss, add=True)

        def scatter_wait(j, sb, ss):
            pltpu.make_async_copy(sb, acc.at[didx.at[j]], ss).wait()

        gather(0, rb0, gs0)
        gather(1, rb1, gs1)
        gather_wait(0, rb0, gs0)
        scale(0, rb0, sb0)
        scatter(0, sb0, ss)
        gather(2, rb0, gs0)
        gather_wait(1, rb1, gs1)
        scale(1, rb1, sb1)
        scatter(1, sb1, ss)
        gather(3, rb1, gs1)

        def pair(jj, carry):
            j0 = 2 * jj
            gather_wait(j0, rb0, gs0)
            scatter_wait(j0 - 2, sb0, ss)
            scale(j0, rb0, sb0)
            scatter(j0, sb0, ss)

            @pl.when(j0 + 2 < cpt)
            def _():
                gather(j0 + 2, rb0, gs0)

            gather_wait(j0 + 1, rb1, gs1)
            scatter_wait(j0 - 1, sb1, ss)
            scale(j0 + 1, rb1, sb1)
            scatter(j0 + 1, sb1, ss)

            @pl.when(j0 + 3 < cpt)
            def _():
                gather(j0 + 3, rb1, gs1)

            return carry

        lax.fori_loop(1, cpt // 2, pair, 0)
        scatter_wait(cpt - 2, sb0, ss)
        scatter_wait(cpt - 1, sb1, ss)
        plsc.subcore_barrier()
        pltpu.sync_copy(acc.at[pl.ds(s * rows, rows)],
                        out_hbm.at[c, pl.ds(s * rows, rows)])

    return k


def _make_sc_mp1(NP, EP, ck):
    cpt = EP // (NW * ck)
    assert cpt % 2 == 0
    rows = NP // NS

    @functools.partial(
        pl.kernel, mesh=_mesh, compiler_params=_sc_params,
        out_type=jax.ShapeDtypeStruct((NC, NP), jnp.float32),
        scratch_types=[
            pltpu.VMEM((cpt, ck), jnp.int32),
            pltpu.VMEM((cpt, ck), jnp.int32),
            pltpu.VMEM((cpt, ck), jnp.float32),
            pltpu.VMEM((ck,), jnp.float32),
            pltpu.VMEM((ck,), jnp.float32),
            pltpu.VMEM((rows,), jnp.float32),
            pltpu.VMEM_SHARED((NP,), jnp.float32),
            pltpu.SemaphoreType.DMA,
            pltpu.SemaphoreType.DMA,
        ],
    )
    def k(g_hbm, src_hbm, dst_hbm, w_hbm, z_hbm, out_hbm,
          sidx, didx, wv, vb0, vb1, stage, acc, sem0, sem1):
        c = lax.axis_index("c")
        s = lax.axis_index("s")
        cbase = (c * NS + s) * cpt
        pltpu.sync_copy(z_hbm, stage)
        pltpu.sync_copy(stage, acc.at[pl.ds(s * rows, rows)])
        pltpu.sync_copy(src_hbm.at[pl.ds(cbase, cpt)], sidx)
        pltpu.sync_copy(dst_hbm.at[pl.ds(cbase, cpt)], didx)
        pltpu.sync_copy(w_hbm.at[pl.ds(cbase, cpt)], wv)
        plsc.subcore_barrier()

        def scale_scatter(j, vb):
            for i in range(ck // LN):
                sl = pl.ds(i * LN, LN)
                vb[sl] = vb[sl] * wv[j, sl]
            pltpu.sync_copy(vb, acc.at[didx.at[j]], add=True)

        pltpu.async_copy(g_hbm.at[sidx.at[0]], vb0, sem0)

        def pair(jj, carry):
            j0 = 2 * jj
            pltpu.make_async_copy(g_hbm.at[sidx.at[j0]], vb0, sem0).wait()
            pltpu.async_copy(g_hbm.at[sidx.at[j0 + 1]], vb1, sem1)
            scale_scatter(j0, vb0)
            pltpu.make_async_copy(g_hbm.at[sidx.at[j0 + 1]], vb1, sem1).wait()

            @pl.when(jj + 1 < cpt // 2)
            def _():
                pltpu.async_copy(g_hbm.at[sidx.at[j0 + 2]], vb0, sem0)

            scale_scatter(j0 + 1, vb1)
            return carry

        lax.fori_loop(0, cpt // 2, pair, 0)
        plsc.subcore_barrier()
        pltpu.sync_copy(acc.at[pl.ds(s * rows, rows)], stage)
        pltpu.sync_copy(stage, out_hbm.at[c, pl.ds(s * rows, rows)])

    return k


def _tc_first(NP, D, F):
    def body(deg_ref, x_ref, w_ref, dis_ref, g_ref):
        deg = deg_ref[0] + deg_ref[1] + 1.0
        dis = lax.rsqrt(deg)
        dis_ref[...] = dis
        g_ref[...] = jnp.dot(x_ref[...], w_ref[...],
                             preferred_element_type=jnp.float32) * dis

    return pl.pallas_call(
        body,
        grid=(NP // RB,),
        in_specs=[
            pl.BlockSpec((NC, RB, 1), lambda i: (0, i, 0)),
            pl.BlockSpec((RB, D), lambda i: (i, 0)),
            pl.BlockSpec((D, F), lambda i: (0, 0)),
        ],
        out_specs=[
            pl.BlockSpec((RB, 1), lambda i: (i, 0)),
            pl.BlockSpec((RB, F), lambda i: (i, 0)),
        ],
        out_shape=[
            jax.ShapeDtypeStruct((NP, 1), jnp.float32),
            jax.ShapeDtypeStruct((NP, F), jnp.float32),
        ],
    )


def _tc_mid(NP, Fin, Fout):
    split = Fout == 128

    def body(s_ref, g_ref, dis_ref, b_ref, w_ref, *out_refs):
        dis = dis_ref[...]
        h = dis * (s_ref[0] + s_ref[1] + g_ref[...]) + b_ref[...]
        h = jnp.maximum(h, 0.0)
        g = jnp.dot(h, w_ref[...], preferred_element_type=jnp.float32) * dis
        if split:
            out_refs[0][...] = g[:, :64]
            out_refs[1][...] = g[:, 64:]
        else:
            out_refs[0][...] = g

    if split:
        out_specs = [pl.BlockSpec((RB, 64), lambda i: (i, 0)),
                     pl.BlockSpec((RB, 64), lambda i: (i, 0))]
        out_shape = [jax.ShapeDtypeStruct((NP, 64), jnp.float32),
                     jax.ShapeDtypeStruct((NP, 64), jnp.float32)]
    else:
        out_specs = pl.BlockSpec((RB, Fout), lambda i: (i, 0))
        out_shape = jax.ShapeDtypeStruct((NP, Fout), jnp.float32)

    return pl.pallas_call(
        body,
        grid=(NP // RB,),
        in_specs=[
            pl.BlockSpec((NC, RB, Fin), lambda i: (0, i, 0)),
            pl.BlockSpec((RB, Fin), lambda i: (i, 0)),
            pl.BlockSpec((RB, 1), lambda i: (i, 0)),
            pl.BlockSpec((1, Fin), lambda i: (0, 0)),
            pl.BlockSpec((Fin, Fout), lambda i: (0, 0)),
        ],
        out_specs=out_specs,
        out_shape=out_shape,
    )


def _tc_mid2(NP, Fout):
    def body(sa_ref, sb_ref, ga_ref, gb_ref, dis_ref, b_ref, w_ref, out_ref):
        dis = dis_ref[...]
        ha = dis * (sa_ref[0] + sa_ref[1] + ga_ref[...])
        hb = dis * (sb_ref[0] + sb_ref[1] + gb_ref[...])
        h = jnp.concatenate([ha, hb], axis=1) + b_ref[...]
        h = jnp.maximum(h, 0.0)
        out_ref[...] = jnp.dot(h, w_ref[...],
                               preferred_element_type=jnp.float32) * dis

    return pl.pallas_call(
        body,
        grid=(NP // RB,),
        in_specs=[
            pl.BlockSpec((NC, RB, 64), lambda i: (0, i, 0)),
            pl.BlockSpec((NC, RB, 64), lambda i: (0, i, 0)),
            pl.BlockSpec((RB, 64), lambda i: (i, 0)),
            pl.BlockSpec((RB, 64), lambda i: (i, 0)),
            pl.BlockSpec((RB, 1), lambda i: (i, 0)),
            pl.BlockSpec((1, 128), lambda i: (0, 0)),
            pl.BlockSpec((128, Fout), lambda i: (0, 0)),
        ],
        out_specs=pl.BlockSpec((RB, Fout), lambda i: (i, 0)),
        out_shape=jax.ShapeDtypeStruct((NP, Fout), jnp.float32),
    )


def _tc_last(NP, G):
    def body(s_ref, g_ref, dis_ref, b_ref, batch_ref, out_ref):
        h = dis_ref[...] * (s_ref[0] + s_ref[1] + g_ref[...]) + b_ref[...]
        gids = lax.broadcasted_iota(jnp.int32, (1, G), 1)
        onehot = (batch_ref[...] == gids).astype(jnp.float32)
        dn = (((0,), (0,)), ((), ()))
        sums = lax.dot_general(onehot, h, dn,
                               preferred_element_type=jnp.float32)
        ones = jnp.ones((NP, 1), jnp.float32)
        cnts = lax.dot_general(onehot, ones, dn,
                               preferred_element_type=jnp.float32)
        out_ref[...] = sums / jnp.maximum(cnts, 1.0)

    return pl.pallas_call(
        body,
        in_specs=[
            pl.BlockSpec((NC, NP, 1), lambda: (0, 0, 0)),
            pl.BlockSpec((NP, 1), lambda: (0, 0)),
            pl.BlockSpec((NP, 1), lambda: (0, 0)),
            pl.BlockSpec((1, 1), lambda: (0, 0)),
            pl.BlockSpec((NP, 1), lambda: (0, 0)),
        ],
        out_specs=pl.BlockSpec((G, 1), lambda: (0, 0)),
        out_shape=jax.ShapeDtypeStruct((G, 1), jnp.float32),
    )


def kernel(x, edge_index, edge_weight, batch,
           W1, b1, W2, b2, W3, b3, W4, b4):
    N, D = x.shape
    E = edge_index.shape[1]
    G = NGROUPS
    f32 = jnp.float32

    NP = ((N + 1 + RB - 1) // RB) * RB
    EP = -(-E // (2 * NW * CKS)) * (2 * NW * CKS)
    rows = NP // NS

    pad_dst = N + (jnp.arange(EP - E, dtype=jnp.int32) % (NP - N))
    pad_src = jnp.arange(EP - E, dtype=jnp.int32) % N
    src_f = jnp.concatenate([edge_index[0], pad_src])
    dst_f = jnp.concatenate([edge_index[1], pad_dst])
    w_f = jnp.concatenate([edge_weight, jnp.zeros((EP - E,), f32)])
    src_w = src_f.reshape(EP // CKW, CKW)
    dst_w = dst_f.reshape(EP // CKW, CKW)
    w_w = w_f.reshape(EP // CKW, CKW)
    src_s = src_f.reshape(EP // CKS, CKS)
    dst_s = dst_f.reshape(EP // CKS, CKS)
    w_s = w_f.reshape(EP // CKS, CKS)
    x_p = jnp.concatenate([x, jnp.zeros((NP - N, D), f32)])
    batch_p = jnp.concatenate([batch, jnp.full((NP - N,), G, jnp.int32)])[:, None]
    z1 = jnp.zeros((rows,), f32)
    z64 = jnp.zeros((rows, 64), f32)

    sc_deg = _make_sc_deg(NP, EP, CKS)
    sc_mp64 = _make_sc_mp(NP, EP, 64, CKW)
    sc_mp1 = _make_sc_mp1(NP, EP, CKS)

    deg_part = sc_deg(dst_s, w_s, z1)
    dis, g1 = _tc_first(NP, D, 64)(deg_part[..., None], x_p, W1)
    s1 = sc_mp64(g1, src_w, dst_w, w_w, z64)
    g2a, g2b = _tc_mid(NP, 64, 128)(s1, g1, dis, b1[None, :], W2)
    s2a = sc_mp64(g2a, src_w, dst_w, w_w, z64)
    s2b = sc_mp64(g2b, src_w, dst_w, w_w, z64)
    g3 = _tc_mid2(NP, 64)(s2a, s2b, g2a, g2b, dis, b2[None, :], W3)
    s3 = sc_mp64(g3, src_w, dst_w, w_w, z64)
    g4 = _tc_mid(NP, 64, 1)(s3, g3, dis, b3[None, :], W4)
    s4 = sc_mp1(g4[:, 0], src_s, dst_s, w_s, z1)
    out = _tc_last(NP, G)(s4[..., None], g4, dis, b4[None, :], batch_p)
    return out


# speedup vs baseline: 1.0145x; 1.0064x over previous
import functools
import jax
import jax.numpy as jnp
from jax import lax
from jax.experimental import pallas as pl
from jax.experimental.pallas import tpu as pltpu
from jax.experimental.pallas import tpu_sc as plsc

NC = 2
NS = 16
LN = 16
NW = NC * NS
CKW = 128
CKS = 512
STG = 128
RB = 1024
NGROUPS = 16

_mesh = plsc.VectorSubcoreMesh(
    core_axis_name="c", subcore_axis_name="s", num_cores=NC, num_subcores=NS)
_sc_params = pltpu.CompilerParams(use_tc_tiling_on_sc=False)


def _make_sc_deg(NP, EP, ck):
    cpt = EP // (NW * ck)
    rows = NP // NS

    @functools.partial(
        pl.kernel, mesh=_mesh, compiler_params=_sc_params,
        out_type=jax.ShapeDtypeStruct((NC, NP), jnp.float32),
        scratch_types=[
            pltpu.VMEM((cpt, ck), jnp.int32),
            pltpu.VMEM((cpt, ck), jnp.float32),
            pltpu.VMEM((rows,), jnp.float32),
            pltpu.VMEM_SHARED((NP,), jnp.float32),
        ],
    )
    def k(dst_hbm, w_hbm, z_hbm, out_hbm, didx, wv, stage, acc):
        c = lax.axis_index("c")
        s = lax.axis_index("s")
        cbase = (c * NS + s) * cpt
        pltpu.sync_copy(z_hbm, stage)
        pltpu.sync_copy(stage, acc.at[pl.ds(s * rows, rows)])
        pltpu.sync_copy(dst_hbm.at[pl.ds(cbase, cpt)], didx)
        pltpu.sync_copy(w_hbm.at[pl.ds(cbase, cpt)], wv)
        plsc.subcore_barrier()

        def chunk(j, carry):
            pltpu.sync_copy(wv.at[j], acc.at[didx.at[j]], add=True)
            return carry

        lax.fori_loop(0, cpt, chunk, 0)
        plsc.subcore_barrier()
        pltpu.sync_copy(acc.at[pl.ds(s * rows, rows)], stage)
        pltpu.sync_copy(stage, out_hbm.at[c, pl.ds(s * rows, rows)])

    return k


def _make_sc_mp(NP, EP, F, ck):
    cpt = EP // (NW * ck)
    assert cpt % 2 == 0
    rows = NP // NS

    @functools.partial(
        pl.kernel, mesh=_mesh, compiler_params=_sc_params,
        out_type=jax.ShapeDtypeStruct((NC, NP, F), jnp.float32),
        scratch_types=[
            pltpu.VMEM((cpt, ck), jnp.int32),
            pltpu.VMEM((cpt, ck), jnp.int32),
            pltpu.VMEM((cpt, ck), jnp.float32),
            pltpu.VMEM((ck, F), jnp.float32),
            pltpu.VMEM((ck, F), jnp.float32),
            pltpu.VMEM((ck, F), jnp.float32),
            pltpu.VMEM((ck, F), jnp.float32),
            pltpu.VMEM((STG, F), jnp.float32),
            pltpu.VMEM_SHARED((NP, F), jnp.float32),
            pltpu.SemaphoreType.DMA,
            pltpu.SemaphoreType.DMA,
            pltpu.SemaphoreType.DMA,
        ],
    )
    def k(g_hbm, src_hbm, dst_hbm, w_hbm, z_hbm, out_hbm,
          sidx, didx, wv, rb0, rb1, sb0, sb1, stage, acc,
          gs0, gs1, ss):
        c = lax.axis_index("c")
        s = lax.axis_index("s")
        cbase = (c * NS + s) * cpt
        pltpu.sync_copy(z_hbm, stage)
        for r in range(rows // STG):
            pltpu.sync_copy(stage, acc.at[pl.ds(s * rows + r * STG, STG)])
        pltpu.sync_copy(src_hbm.at[pl.ds(cbase, cpt)], sidx)
        pltpu.sync_copy(dst_hbm.at[pl.ds(cbase, cpt)], didx)
        pltpu.sync_copy(w_hbm.at[pl.ds(cbase, cpt)], wv)
        plsc.subcore_barrier()

        def gather(j, rb, gs):
            pltpu.async_copy(g_hbm.at[sidx.at[j]], rb, gs)

        def gather_wait(j, rb, gs):
            pltpu.make_async_copy(g_hbm.at[sidx.at[j]], rb, gs).wait()

        def scale(j, rb, sb):
            def egrp(i, ecarry):
                wv16 = wv[j, pl.ds(i * LN, LN)]
                for j16 in range(LN):
                    bw = jnp.full((LN,), wv16[j16], jnp.float32)
                    e = i * LN + j16
                    for f in range(F // LN):
                        sl = pl.ds(f * LN, LN)
                        sb[e, sl] = rb[e, sl] * bw
                return ecarry

            lax.fori_loop(0, ck // LN, egrp, 0)

        def scatter(j, sb, ss):
            pltpu.async_copy(sb, acc.at[didx.at[j]], ss, add=True)

        def scatter_wait(j, sb, ss):
            pltpu.make_async_copy(sb, acc.at[didx.at[j]], ss).wait()

        gather(0, rb0, gs0)
        gather(1, rb1, gs1)
        gather_wait(0, rb0, gs0)
        scale(0, rb0, sb0)
        scatter(0, sb0, ss)
        gather(2, rb0, gs0)
        gather_wait(1, rb1, gs1)
        scale(1, rb1, sb1)
        scatter(1, sb1, ss)
        gather(3, rb1, gs1)

        def pair(jj, carry):
            j0 = 2 * jj
            gather_wait(j0, rb0, gs0)
            scatter_wait(j0 - 2, sb0, ss)
            scale(j0, rb0, sb0)
            scatter(j0, sb0, ss)

            @pl.when(j0 + 2 < cpt)
            def _():
                gather(j0 + 2, rb0, gs0)

            gather_wait(j0 + 1, rb1, gs1)
            scatter_wait(j0 - 1, sb1, ss)
            scale(j0 + 1, rb1, sb1)
            scatter(j0 + 1, sb1, ss)

            @pl.when(j0 + 3 < cpt)
            def _():
                gather(j0 + 3, rb1, gs1)

            return carry

        lax.fori_loop(1, cpt // 2, pair, 0)
        scatter_wait(cpt - 2, sb0, ss)
        scatter_wait(cpt - 1, sb1, ss)
        plsc.subcore_barrier()
        for r in range(rows // STG):
            pltpu.sync_copy(acc.at[pl.ds(s * rows + r * STG, STG)], stage)
            pltpu.sync_copy(stage, out_hbm.at[c, pl.ds(s * rows + r * STG, STG)])

    return k


def _make_sc_mp1(NP, EP, ck):
    cpt = EP // (NW * ck)
    assert cpt % 2 == 0
    rows = NP // NS

    @functools.partial(
        pl.kernel, mesh=_mesh, compiler_params=_sc_params,
        out_type=jax.ShapeDtypeStruct((NC, NP), jnp.float32),
        scratch_types=[
            pltpu.VMEM((cpt, ck), jnp.int32),
            pltpu.VMEM((cpt, ck), jnp.int32),
            pltpu.VMEM((cpt, ck), jnp.float32),
            pltpu.VMEM((ck,), jnp.float32),
            pltpu.VMEM((ck,), jnp.float32),
            pltpu.VMEM((rows,), jnp.float32),
            pltpu.VMEM_SHARED((NP,), jnp.float32),
            pltpu.SemaphoreType.DMA,
            pltpu.SemaphoreType.DMA,
        ],
    )
    def k(g_hbm, src_hbm, dst_hbm, w_hbm, z_hbm, out_hbm,
          sidx, didx, wv, vb0, vb1, stage, acc, sem0, sem1):
        c = lax.axis_index("c")
        s = lax.axis_index("s")
        cbase = (c * NS + s) * cpt
        pltpu.sync_copy(z_hbm, stage)
        pltpu.sync_copy(stage, acc.at[pl.ds(s * rows, rows)])
        pltpu.sync_copy(src_hbm.at[pl.ds(cbase, cpt)], sidx)
        pltpu.sync_copy(dst_hbm.at[pl.ds(cbase, cpt)], didx)
        pltpu.sync_copy(w_hbm.at[pl.ds(cbase, cpt)], wv)
        plsc.subcore_barrier()

        def scale_scatter(j, vb):
            for i in range(ck // LN):
                sl = pl.ds(i * LN, LN)
                vb[sl] = vb[sl] * wv[j, sl]
            pltpu.sync_copy(vb, acc.at[didx.at[j]], add=True)

        pltpu.async_copy(g_hbm.at[sidx.at[0]], vb0, sem0)

        def pair(jj, carry):
            j0 = 2 * jj
            pltpu.make_async_copy(g_hbm.at[sidx.at[j0]], vb0, sem0).wait()
            pltpu.async_copy(g_hbm.at[sidx.at[j0 + 1]], vb1, sem1)
            scale_scatter(j0, vb0)
            pltpu.make_async_copy(g_hbm.at[sidx.at[j0 + 1]], vb1, sem1).wait()

            @pl.when(jj + 1 < cpt // 2)
            def _():
                pltpu.async_copy(g_hbm.at[sidx.at[j0 + 2]], vb0, sem0)

            scale_scatter(j0 + 1, vb1)
            return carry

        lax.fori_loop(0, cpt // 2, pair, 0)
        plsc.subcore_barrier()
        pltpu.sync_copy(acc.at[pl.ds(s * rows, rows)], stage)
        pltpu.sync_copy(stage, out_hbm.at[c, pl.ds(s * rows, rows)])

    return k


def _tc_first(NP, D, F):
    def body(deg_ref, x_ref, w_ref, dis_ref, g_ref):
        deg = deg_ref[0] + deg_ref[1] + 1.0
        dis = lax.rsqrt(deg)
        dis_ref[...] = dis
        g_ref[...] = jnp.dot(x_ref[...], w_ref[...],
                             preferred_element_type=jnp.float32) * dis

    return pl.pallas_call(
        body,
        grid=(NP // RB,),
        in_specs=[
            pl.BlockSpec((NC, RB, 1), lambda i: (0, i, 0)),
            pl.BlockSpec((RB, D), lambda i: (i, 0)),
            pl.BlockSpec((D, F), lambda i: (0, 0)),
        ],
        out_specs=[
            pl.BlockSpec((RB, 1), lambda i: (i, 0)),
            pl.BlockSpec((RB, F), lambda i: (i, 0)),
        ],
        out_shape=[
            jax.ShapeDtypeStruct((NP, 1), jnp.float32),
            jax.ShapeDtypeStruct((NP, F), jnp.float32),
        ],
    )


def _tc_mid(NP, Fin, Fout):
    split = Fout == 128

    def body(s_ref, g_ref, dis_ref, b_ref, w_ref, *out_refs):
        dis = dis_ref[...]
        h = dis * (s_ref[0] + s_ref[1] + g_ref[...]) + b_ref[...]
        h = jnp.maximum(h, 0.0)
        g = jnp.dot(h, w_ref[...], preferred_element_type=jnp.float32) * dis
        if split:
            out_refs[0][...] = g[:, :64]
            out_refs[1][...] = g[:, 64:]
        else:
            out_refs[0][...] = g

    if split:
        out_specs = [pl.BlockSpec((RB, 64), lambda i: (i, 0)),
                     pl.BlockSpec((RB, 64), lambda i: (i, 0))]
        out_shape = [jax.ShapeDtypeStruct((NP, 64), jnp.float32),
                     jax.ShapeDtypeStruct((NP, 64), jnp.float32)]
    else:
        out_specs = pl.BlockSpec((RB, Fout), lambda i: (i, 0))
        out_shape = jax.ShapeDtypeStruct((NP, Fout), jnp.float32)

    return pl.pallas_call(
        body,
        grid=(NP // RB,),
        in_specs=[
            pl.BlockSpec((NC, RB, Fin), lambda i: (0, i, 0)),
            pl.BlockSpec((RB, Fin), lambda i: (i, 0)),
            pl.BlockSpec((RB, 1), lambda i: (i, 0)),
            pl.BlockSpec((1, Fin), lambda i: (0, 0)),
            pl.BlockSpec((Fin, Fout), lambda i: (0, 0)),
        ],
        out_specs=out_specs,
        out_shape=out_shape,
    )


def _tc_mid2(NP, Fout):
    def body(sa_ref, sb_ref, ga_ref, gb_ref, dis_ref, b_ref, w_ref, out_ref):
        dis = dis_ref[...]
        ha = dis * (sa_ref[0] + sa_ref[1] + ga_ref[...])
        hb = dis * (sb_ref[0] + sb_ref[1] + gb_ref[...])
        h = jnp.concatenate([ha, hb], axis=1) + b_ref[...]
        h = jnp.maximum(h, 0.0)
        out_ref[...] = jnp.dot(h, w_ref[...],
                               preferred_element_type=jnp.float32) * dis

    return pl.pallas_call(
        body,
        grid=(NP // RB,),
        in_specs=[
            pl.BlockSpec((NC, RB, 64), lambda i: (0, i, 0)),
            pl.BlockSpec((NC, RB, 64), lambda i: (0, i, 0)),
            pl.BlockSpec((RB, 64), lambda i: (i, 0)),
            pl.BlockSpec((RB, 64), lambda i: (i, 0)),
            pl.BlockSpec((RB, 1), lambda i: (i, 0)),
            pl.BlockSpec((1, 128), lambda i: (0, 0)),
            pl.BlockSpec((128, Fout), lambda i: (0, 0)),
        ],
        out_specs=pl.BlockSpec((RB, Fout), lambda i: (i, 0)),
        out_shape=jax.ShapeDtypeStruct((NP, Fout), jnp.float32),
    )


def _tc_last(NP, G):
    def body(s_ref, g_ref, dis_ref, b_ref, batch_ref, out_ref):
        h = dis_ref[...] * (s_ref[0] + s_ref[1] + g_ref[...]) + b_ref[...]
        gids = lax.broadcasted_iota(jnp.int32, (1, G), 1)
        onehot = (batch_ref[...] == gids).astype(jnp.float32)
        dn = (((0,), (0,)), ((), ()))
        sums = lax.dot_general(onehot, h, dn,
                               preferred_element_type=jnp.float32)
        ones = jnp.ones((NP, 1), jnp.float32)
        cnts = lax.dot_general(onehot, ones, dn,
                               preferred_element_type=jnp.float32)
        out_ref[...] = sums / jnp.maximum(cnts, 1.0)

    return pl.pallas_call(
        body,
        in_specs=[
            pl.BlockSpec((NC, NP, 1), lambda: (0, 0, 0)),
            pl.BlockSpec((NP, 1), lambda: (0, 0)),
            pl.BlockSpec((NP, 1), lambda: (0, 0)),
            pl.BlockSpec((1, 1), lambda: (0, 0)),
            pl.BlockSpec((NP, 1), lambda: (0, 0)),
        ],
        out_specs=pl.BlockSpec((G, 1), lambda: (0, 0)),
        out_shape=jax.ShapeDtypeStruct((G, 1), jnp.float32),
    )


def kernel(x, edge_index, edge_weight, batch,
           W1, b1, W2, b2, W3, b3, W4, b4):
    N, D = x.shape
    E = edge_index.shape[1]
    G = NGROUPS
    f32 = jnp.float32

    NP = ((N + 1 + RB - 1) // RB) * RB
    EP = -(-E // (2 * NW * CKS)) * (2 * NW * CKS)
    rows = NP // NS

    pad_dst = N + (jnp.arange(EP - E, dtype=jnp.int32) % (NP - N))
    pad_src = jnp.arange(EP - E, dtype=jnp.int32) % N
    src_f = jnp.concatenate([edge_index[0], pad_src])
    dst_f = jnp.concatenate([edge_index[1], pad_dst])
    w_f = jnp.concatenate([edge_weight, jnp.zeros((EP - E,), f32)])
    src_w = src_f.reshape(EP // CKW, CKW)
    dst_w = dst_f.reshape(EP // CKW, CKW)
    w_w = w_f.reshape(EP // CKW, CKW)
    src_s = src_f.reshape(EP // CKS, CKS)
    dst_s = dst_f.reshape(EP // CKS, CKS)
    w_s = w_f.reshape(EP // CKS, CKS)
    x_p = jnp.concatenate([x, jnp.zeros((NP - N, D), f32)])
    batch_p = jnp.concatenate([batch, jnp.full((NP - N,), G, jnp.int32)])[:, None]
    z1 = jnp.zeros((rows,), f32)
    z64 = jnp.zeros((STG, 64), f32)

    sc_deg = _make_sc_deg(NP, EP, CKS)
    sc_mp64 = _make_sc_mp(NP, EP, 64, CKW)
    sc_mp1 = _make_sc_mp1(NP, EP, CKS)

    deg_part = sc_deg(dst_s, w_s, z1)
    dis, g1 = _tc_first(NP, D, 64)(deg_part[..., None], x_p, W1)
    s1 = sc_mp64(g1, src_w, dst_w, w_w, z64)
    g2a, g2b = _tc_mid(NP, 64, 128)(s1, g1, dis, b1[None, :], W2)
    s2a = sc_mp64(g2a, src_w, dst_w, w_w, z64)
    s2b = sc_mp64(g2b, src_w, dst_w, w_w, z64)
    g3 = _tc_mid2(NP, 64)(s2a, s2b, g2a, g2b, dis, b2[None, :], W3)
    s3 = sc_mp64(g3, src_w, dst_w, w_w, z64)
    g4 = _tc_mid(NP, 64, 1)(s3, g3, dis, b3[None, :], W4)
    s4 = sc_mp1(g4[:, 0], src_s, dst_s, w_s, z1)
    out = _tc_last(NP, G)(s4[..., None], g4, dis, b4[None, :], batch_p)
    return out
